# baseline (speedup 1.0000x reference)
import functools

import jax
import jax.numpy as jnp
from jax import lax
from jax.experimental import pallas as pl
from jax.experimental.pallas import tpu as pltpu
from jax.experimental.pallas import tpu_sc as plsc

D_MODEL = 768
D_LATENT = 12288
TOPK = 20
N_TOKENS = 8192
NGRP = 96
NSEL = 24
NSEL_PAD = 40
BR_ENC = 256
BC_ENC = 2048
GPB = BC_ENC // 128
BR_SEL = 512

NC = 2
NS = 16
NW = NC * NS
RPW = N_TOKENS // NW
CMAX = NSEL * 128 + 16


def _enc_body(x_ref, we_ref, be_ref, lat_ref, m_ref):
    acc = jnp.dot(x_ref[...].astype(jnp.bfloat16),
                  we_ref[...].astype(jnp.bfloat16),
                  preferred_element_type=jnp.float32)
    latf = acc + be_ref[...]
    lat_ref[...] = latf
    cols = [jnp.max(latf[:, g * 128:(g + 1) * 128], axis=1, keepdims=True)
            for g in range(GPB)]
    m_ref[...] = jnp.concatenate(cols, axis=1)[None]


def _sel_body(m_ref, gi_ref, lb_ref):
    i = pl.program_id(0)
    work = m_ref[...]
    iota = lax.broadcasted_iota(jnp.int32, work.shape, 1)
    rowbase = (i * BR_SEL
               + lax.broadcasted_iota(jnp.int32, (BR_SEL, 1), 0)) * NGRP
    m = None
    for j in range(NSEL):
        m = jnp.max(work, axis=1, keepdims=True)
        g = jnp.min(jnp.where(work >= m, iota, 2 ** 30), axis=1,
                    keepdims=True)
        gi_ref[:, j:j + 1] = rowbase + g
        work = jnp.where(iota == g, -jnp.inf, work)
    gi_ref[:, NSEL:NSEL_PAD] = jnp.zeros((BR_SEL, NSEL_PAD - NSEL), jnp.int32)
    lb_ref[...] = m


def _sc_body(lat_hbm, gidx_hbm, lb_hbm, wd_hbm, bd_hbm, out_hbm,
             gidx_v, lb_v, grp_v, cw_v, cv_v, ci_v, sv_v, si_v, gbuf_v,
             wrow_v, bd_v, orow_v, sem1, sem2):
    wid = lax.axis_index("s") * NC + lax.axis_index("c")
    base = wid * RPW
    pltpu.sync_copy(gidx_hbm.at[pl.ds(base, RPW)], gidx_v)
    pltpu.sync_copy(lb_hbm.at[pl.ds(base, RPW)], lb_v.at[pl.ds(0, RPW)])
    pltpu.sync_copy(bd_hbm, bd_v)
    zeros16i = jnp.zeros((16,), jnp.int32)
    si_v[pl.ds(0, 16)] = zeros16i
    si_v[pl.ds(16, 16)] = zeros16i
    neg16 = jnp.full((16,), -jnp.inf, jnp.float32)
    iota16 = lax.iota(jnp.int32, 16)

    def row_body(r, _):
        pltpu.async_copy(lat_hbm.at[gidx_v.at[r, pl.ds(0, NSEL)]], grp_v,
                         sem1).wait()
        lbv = jnp.full((16,), lb_v[pl.ds(r, 16)][0], jnp.float32)

        def scan_g(g, cpos):
            gid = jnp.full((16,), gidx_v[r, pl.ds(g, 16)][0], jnp.int32)
            colbase = (gid % NGRP) * 128
            for v in range(8):
                xv = grp_v[g, pl.ds(v * 16, 16)]
                colv = colbase + (v * 16) + iota16
                msk = xv >= lbv
                cs = plsc.cumsum(msk.astype(jnp.int32))
                tgt = cpos + cs - 1
                plsc.store_scatter(cw_v, [tgt], xv, mask=msk)
                plsc.store_scatter(cv_v, [tgt], xv, mask=msk)
                plsc.store_scatter(ci_v, [tgt], colv, mask=msk)
                cpos = cpos + jnp.max(cs)
            return cpos

        cpos = lax.fori_loop(0, NSEL, scan_g, jnp.int32(0))
        cw_v[pl.ds(cpos, 16)] = neg16
        nv = (cpos + 15) // 16

        def ext_body(j, _):
            def mx(v, acc):
                return jnp.maximum(acc, cw_v[pl.ds(v * 16, 16)])
            run = lax.fori_loop(0, nv, mx, neg16)
            m = jnp.max(run)
            msp = jnp.full((16,), m, jnp.float32)

            def rm(v, c):
                xv = cw_v[pl.ds(v * 16, 16)]
                cw_v[pl.ds(v * 16, 16)] = jnp.where(xv >= msp, -jnp.inf, xv)
                return c
            lax.fori_loop(0, nv, rm, jnp.int32(0))
            return m

        tau = lax.fori_loop(0, TOPK, ext_body, jnp.float32(0.0))
        tsp = jnp.full((16,), tau, jnp.float32)

        def selp(v, spos):
            xv = cv_v[pl.ds(v * 16, 16)]
            iv = ci_v[pl.ds(v * 16, 16)]
            msk = xv >= tsp
            cs = plsc.cumsum(msk.astype(jnp.int32))
            tgt = spos + cs - 1
            plsc.store_scatter(sv_v, [tgt], xv, mask=msk)
            plsc.store_scatter(si_v, [tgt], iv, mask=msk)
            return spos + jnp.max(cs)

        lax.fori_loop(0, nv, selp, jnp.int32(0))

        gbuf_v[pl.ds(0, 16)] = si_v[pl.ds(0, 16)]
        gbuf_v[pl.ds(8, 16)] = si_v[pl.ds(8, 16)]
        pltpu.async_copy(wd_hbm.at[gbuf_v], wrow_v, sem2).wait()

        for v in range(D_MODEL // 16):
            orow_v[pl.ds(v * 16, 16)] = bd_v[pl.ds(v * 16, 16)]

        def dec_body(j, c):
            valv = jnp.full((16,), sv_v[pl.ds(j, 16)][0], jnp.float32)
            for v in range(D_MODEL // 16):
                wv = wrow_v[j, pl.ds(v * 16, 16)]
                plsc.addupdate(orow_v.at[pl.ds(v * 16, 16)], valv * wv)
            return c

        lax.fori_loop(0, TOPK, dec_body, jnp.int32(0))
        pltpu.sync_copy(orow_v, out_hbm.at[base + r])
        return 0

    lax.fori_loop(0, RPW, row_body, jnp.int32(0))


@jax.jit
def kernel(x, W_enc, b_enc, W_dec, b_dec):
    be2 = b_enc.reshape(1, D_LATENT)

    latents, M = pl.pallas_call(
        _enc_body,
        grid=(N_TOKENS // BR_ENC, D_LATENT // BC_ENC),
        in_specs=[
            pl.BlockSpec((BR_ENC, D_MODEL), lambda i, j: (i, 0)),
            pl.BlockSpec((D_MODEL, BC_ENC), lambda i, j: (0, j)),
            pl.BlockSpec((1, BC_ENC), lambda i, j: (0, j)),
        ],
        out_specs=[
            pl.BlockSpec((BR_ENC, BC_ENC), lambda i, j: (i, j)),
            pl.BlockSpec((1, BR_ENC, GPB), lambda i, j: (j, i, 0)),
        ],
        out_shape=[
            jax.ShapeDtypeStruct((N_TOKENS, D_LATENT), jnp.float32),
            jax.ShapeDtypeStruct((D_LATENT // BC_ENC, N_TOKENS, GPB),
                             jnp.float32),
        ],
        compiler_params=pltpu.CompilerParams(
            dimension_semantics=("parallel", "parallel")),
    )(x, W_enc, be2)

    M = jnp.transpose(M, (1, 0, 2)).reshape(N_TOKENS, NGRP)

    gidx, lb = pl.pallas_call(
        _sel_body,
        grid=(N_TOKENS // BR_SEL,),
        in_specs=[pl.BlockSpec((BR_SEL, NGRP), lambda i: (i, 0))],
        out_specs=[
            pl.BlockSpec((BR_SEL, NSEL_PAD), lambda i: (i, 0)),
            pl.BlockSpec((BR_SEL, 1), lambda i: (i, 0)),
        ],
        out_shape=[
            jax.ShapeDtypeStruct((N_TOKENS, NSEL_PAD), jnp.int32),
            jax.ShapeDtypeStruct((N_TOKENS, 1), jnp.float32),
        ],
        compiler_params=pltpu.CompilerParams(
            dimension_semantics=("parallel",)),
    )(M)

    lat_flat = latents.reshape(N_TOKENS * NGRP, 128)
    lb_flat = lb.reshape(N_TOKENS)

    sc_fn = functools.partial(
        pl.kernel,
        mesh=plsc.VectorSubcoreMesh(core_axis_name="c",
                                    subcore_axis_name="s"),
        compiler_params=pltpu.CompilerParams(needs_layout_passes=False),
        out_type=jax.ShapeDtypeStruct((N_TOKENS, D_MODEL), jnp.float32),
        scratch_types=[
            pltpu.VMEM((RPW, NSEL_PAD), jnp.int32),
            pltpu.VMEM((RPW + 16,), jnp.float32),
            pltpu.VMEM((NSEL, 128), jnp.float32),
            pltpu.VMEM((CMAX,), jnp.float32),
            pltpu.VMEM((CMAX,), jnp.float32),
            pltpu.VMEM((CMAX,), jnp.int32),
            pltpu.VMEM((48,), jnp.float32),
            pltpu.VMEM((48,), jnp.int32),
            pltpu.VMEM((NSEL,), jnp.int32),
            pltpu.VMEM((NSEL, D_MODEL), jnp.float32),
            pltpu.VMEM((D_MODEL,), jnp.float32),
            pltpu.VMEM((D_MODEL,), jnp.float32),
            pltpu.SemaphoreType.DMA,
            pltpu.SemaphoreType.DMA,
        ],
    )(_sc_body)

    recons = sc_fn(lat_flat, gidx, lb_flat, W_dec, b_dec)
    return recons


# speedup vs baseline: 1.6048x; 1.6048x over previous
import functools

import jax
import jax.numpy as jnp
from jax import lax
from jax.experimental import pallas as pl
from jax.experimental.pallas import tpu as pltpu
from jax.experimental.pallas import tpu_sc as plsc

D_MODEL = 768
D_LATENT = 12288
TOPK = 20
N_TOKENS = 8192
NGRP = 96
NSEL = 24
NSEL_PAD = 40
BR_ENC = 256
BC_ENC = 2048
GPB = BC_ENC // 128
BR_SEL = 512
BR_DEC = 256
BK_DEC = 2048

NC = 2
NS = 16
NW = NC * NS
RPW = N_TOKENS // NW
CMAX = NSEL * 128 + 32


def _enc_body(x_ref, we_ref, be_ref, lat_ref, m_ref):
    acc = jnp.dot(x_ref[...].astype(jnp.bfloat16),
                  we_ref[...].astype(jnp.bfloat16),
                  preferred_element_type=jnp.float32)
    latf = acc + be_ref[...]
    lat_ref[...] = latf
    cols = [jnp.max(latf[:, g * 128:(g + 1) * 128], axis=1, keepdims=True)
            for g in range(GPB)]
    m_ref[...] = jnp.concatenate(cols, axis=1)[None]


def _sel_body(m_ref, gi_ref, lb_ref):
    i = pl.program_id(0)
    work = m_ref[...]
    iota = lax.broadcasted_iota(jnp.int32, work.shape, 1)
    rowbase = (i * BR_SEL
               + lax.broadcasted_iota(jnp.int32, (BR_SEL, 1), 0)) * NGRP
    m = None
    for j in range(NSEL):
        m = jnp.max(work, axis=1, keepdims=True)
        g = jnp.min(jnp.where(work >= m, iota, 2 ** 30), axis=1,
                    keepdims=True)
        gi_ref[:, j:j + 1] = rowbase + g
        work = jnp.where(iota == g, -jnp.inf, work)
    gi_ref[:, NSEL:NSEL_PAD] = jnp.zeros((BR_SEL, NSEL_PAD - NSEL),
                                         jnp.int32)
    lb_ref[...] = m


def _sc_body(lat_hbm, gidx_hbm, lb_hbm, tau_hbm,
             gidx_v, lb_v, grp0_v, grp1_v, cw_v, tau_v, sem0, sem1):
    wid = lax.axis_index("s") * NC + lax.axis_index("c")
    base = wid * RPW
    pltpu.sync_copy(gidx_hbm.at[pl.ds(base, RPW)], gidx_v)
    pltpu.sync_copy(lb_hbm.at[pl.ds(base, RPW)], lb_v.at[pl.ds(0, RPW)])
    neg16 = jnp.full((16,), -jnp.inf, jnp.float32)

    pltpu.async_copy(lat_hbm.at[gidx_v.at[0, pl.ds(0, NSEL)]], grp0_v,
                     sem0)

    def row_body(r, _):
        @pl.when(jnp.logical_and(r + 1 < RPW, lax.rem(r, 2) == 0))
        def _():
            pltpu.async_copy(
                lat_hbm.at[gidx_v.at[r + 1, pl.ds(0, NSEL)]],
                grp1_v, sem1)

        @pl.when(jnp.logical_and(r + 1 < RPW, lax.rem(r, 2) == 1))
        def _():
            pltpu.async_copy(
                lat_hbm.at[gidx_v.at[r + 1, pl.ds(0, NSEL)]],
                grp0_v, sem0)

        lbv = jnp.full((16,), lb_v[pl.ds(r, 16)][0], jnp.float32)

        def scan_one(grp_v, sem):
            pltpu.make_async_copy(
                lat_hbm.at[gidx_v.at[r, pl.ds(0, NSEL)]], grp_v,
                sem).wait()

            def scan_g(g, cpos):
                for v in range(8):
                    xv = grp_v[g, pl.ds(v * 16, 16)]
                    msk = xv >= lbv
                    cs = plsc.cumsum(msk.astype(jnp.int32))
                    tgt = jnp.where(msk, cpos + cs - 1, CMAX - 16)
                    plsc.store_scatter(cw_v, [tgt], xv, mask=msk)
                    cpos = cpos + cs[15]
                return cpos

            return lax.fori_loop(0, NSEL, scan_g, jnp.int32(0))

        cpos = lax.cond(lax.rem(r, 2) == 0,
                        lambda: scan_one(grp0_v, sem0),
                        lambda: scan_one(grp1_v, sem1))
        cw_v[pl.ds(cpos, 16)] = neg16
        nv = (cpos + 15) // 16

        def ext_body(j, _):
            def mx(v, acc):
                return jnp.maximum(acc, cw_v[pl.ds(v * 16, 16)])
            run = lax.fori_loop(0, nv, mx, neg16)
            m = jnp.max(run)
            msp = jnp.full((16,), m, jnp.float32)

            def rm(v, c):
                xv = cw_v[pl.ds(v * 16, 16)]
                cw_v[pl.ds(v * 16, 16)] = jnp.where(xv >= msp, -jnp.inf,
                                                    xv)
                return c
            lax.fori_loop(0, nv, rm, jnp.int32(0))
            return m

        tau = lax.fori_loop(0, TOPK, ext_body, jnp.float32(0.0))
        tau_v[r] = jnp.full((16,), tau, jnp.float32)
        return 0

    lax.fori_loop(0, RPW, row_body, jnp.int32(0))
    pltpu.sync_copy(tau_v, tau_hbm.at[pl.ds(base, RPW)])


def _dec_body(lat_ref, tau_ref, wd_ref, bd_ref, out_ref):
    j = pl.program_id(1)

    @pl.when(j == 0)
    def _():
        out_ref[...] = jnp.broadcast_to(bd_ref[...], out_ref.shape)

    lat = lat_ref[...]
    tau = tau_ref[:, 0:1]
    masked = jnp.where(lat >= tau, lat, 0.0)
    out_ref[...] += jnp.dot(masked.astype(jnp.bfloat16),
                            wd_ref[...].astype(jnp.bfloat16),
                            preferred_element_type=jnp.float32)


@jax.jit
def kernel(x, W_enc, b_enc, W_dec, b_dec):
    be2 = b_enc.reshape(1, D_LATENT)
    bd2 = b_dec.reshape(1, D_MODEL)

    latents, M = pl.pallas_call(
        _enc_body,
        grid=(N_TOKENS // BR_ENC, D_LATENT // BC_ENC),
        in_specs=[
            pl.BlockSpec((BR_ENC, D_MODEL), lambda i, j: (i, 0)),
            pl.BlockSpec((D_MODEL, BC_ENC), lambda i, j: (0, j)),
            pl.BlockSpec((1, BC_ENC), lambda i, j: (0, j)),
        ],
        out_specs=[
            pl.BlockSpec((BR_ENC, BC_ENC), lambda i, j: (i, j)),
            pl.BlockSpec((1, BR_ENC, GPB), lambda i, j: (j, i, 0)),
        ],
        out_shape=[
            jax.ShapeDtypeStruct((N_TOKENS, D_LATENT), jnp.float32),
            jax.ShapeDtypeStruct((D_LATENT // BC_ENC, N_TOKENS, GPB),
                                 jnp.float32),
        ],
        compiler_params=pltpu.CompilerParams(
            dimension_semantics=("parallel", "parallel")),
    )(x, W_enc, be2)

    M = jnp.transpose(M, (1, 0, 2)).reshape(N_TOKENS, NGRP)

    gidx, lb = pl.pallas_call(
        _sel_body,
        grid=(N_TOKENS // BR_SEL,),
        in_specs=[pl.BlockSpec((BR_SEL, NGRP), lambda i: (i, 0))],
        out_specs=[
            pl.BlockSpec((BR_SEL, NSEL_PAD), lambda i: (i, 0)),
            pl.BlockSpec((BR_SEL, 1), lambda i: (i, 0)),
        ],
        out_shape=[
            jax.ShapeDtypeStruct((N_TOKENS, NSEL_PAD), jnp.int32),
            jax.ShapeDtypeStruct((N_TOKENS, 1), jnp.float32),
        ],
        compiler_params=pltpu.CompilerParams(
            dimension_semantics=("parallel",)),
    )(M)

    lat_flat = latents.reshape(N_TOKENS * NGRP, 128)
    lb_flat = lb.reshape(N_TOKENS)

    tau = pl.kernel(
        _sc_body,
        mesh=plsc.VectorSubcoreMesh(core_axis_name="c",
                                    subcore_axis_name="s"),
        compiler_params=pltpu.CompilerParams(needs_layout_passes=False),
        out_type=jax.ShapeDtypeStruct((N_TOKENS, 16), jnp.float32),
        scratch_types=[
            pltpu.VMEM((RPW, NSEL_PAD), jnp.int32),
            pltpu.VMEM((RPW + 16,), jnp.float32),
            pltpu.VMEM((NSEL, 128), jnp.float32),
            pltpu.VMEM((NSEL, 128), jnp.float32),
            pltpu.VMEM((CMAX,), jnp.float32),
            pltpu.VMEM((RPW, 16), jnp.float32),
            pltpu.SemaphoreType.DMA,
            pltpu.SemaphoreType.DMA,
        ],
    )(lat_flat, gidx, lb_flat)

    tau2 = tau[:, 0:1]

    recons = pl.pallas_call(
        _dec_body,
        grid=(N_TOKENS // BR_DEC, D_LATENT // BK_DEC),
        in_specs=[
            pl.BlockSpec((BR_DEC, BK_DEC), lambda i, j: (i, j)),
            pl.BlockSpec((BR_DEC, 1), lambda i, j: (i, 0)),
            pl.BlockSpec((BK_DEC, D_MODEL), lambda i, j: (j, 0)),
            pl.BlockSpec((1, D_MODEL), lambda i, j: (0, 0)),
        ],
        out_specs=pl.BlockSpec((BR_DEC, D_MODEL), lambda i, j: (i, 0)),
        out_shape=jax.ShapeDtypeStruct((N_TOKENS, D_MODEL), jnp.float32),
        compiler_params=pltpu.CompilerParams(
            dimension_semantics=("parallel", "arbitrary")),
    )(latents, tau2, W_dec, bd2)

    return recons


# speedup vs baseline: 1.6900x; 1.0531x over previous
import functools

import jax
import jax.numpy as jnp
from jax import lax
from jax.experimental import pallas as pl
from jax.experimental.pallas import tpu as pltpu
from jax.experimental.pallas import tpu_sc as plsc

D_MODEL = 768
D_LATENT = 12288
TOPK = 20
N_TOKENS = 8192
NGRP = 96
NSEL = 24
NSEL_PAD = 40
BR_ENC = 256
BC_ENC = 2048
GPB = BC_ENC // 128
BR_SEL = 512
BR_DEC = 256
BK_DEC = 2048

NC = 2
NS = 16
NW = NC * NS
RPW = N_TOKENS // NW
CMAX = NSEL * 128 + 32


def _enc_body(x_ref, we_ref, be_ref, lat_ref, m_ref):
    acc = jnp.dot(x_ref[...].astype(jnp.bfloat16),
                  we_ref[...].astype(jnp.bfloat16),
                  preferred_element_type=jnp.float32)
    latf = acc + be_ref[...]
    lat_ref[...] = latf
    cols = [jnp.max(latf[:, g * 128:(g + 1) * 128], axis=1, keepdims=True)
            for g in range(GPB)]
    m_ref[...] = jnp.concatenate(cols, axis=1)[None]


def _sel_body(m_ref, gi_ref, lb_ref):
    i = pl.program_id(0)
    work = m_ref[...]
    iota = lax.broadcasted_iota(jnp.int32, work.shape, 1)
    rowbase = (i * BR_SEL
               + lax.broadcasted_iota(jnp.int32, (BR_SEL, 1), 0)) * NGRP
    m = None
    for j in range(NSEL):
        m = jnp.max(work, axis=1, keepdims=True)
        g = jnp.min(jnp.where(work >= m, iota, 2 ** 30), axis=1,
                    keepdims=True)
        gi_ref[:, j:j + 1] = rowbase + g
        work = jnp.where(iota == g, -jnp.inf, work)
    gi_ref[:, NSEL:NSEL_PAD] = jnp.zeros((BR_SEL, NSEL_PAD - NSEL),
                                         jnp.int32)
    lb_ref[...] = m


def _sc_body(lat_hbm, gidx_hbm, lb_hbm, tau_hbm,
             gidx_v, lb_v, grp0_v, grp1_v, cw_v, tau_v, sem0, sem1):
    wid = lax.axis_index("s") * NC + lax.axis_index("c")
    base = wid * RPW
    pltpu.sync_copy(gidx_hbm.at[pl.ds(base, RPW)], gidx_v)
    pltpu.sync_copy(lb_hbm.at[pl.ds(base, RPW)], lb_v.at[pl.ds(0, RPW)])
    neg16 = jnp.full((16,), -jnp.inf, jnp.float32)

    pltpu.async_copy(lat_hbm.at[gidx_v.at[0, pl.ds(0, NSEL)]], grp0_v,
                     sem0)

    def row_body(r, _):
        @pl.when(jnp.logical_and(r + 1 < RPW, lax.rem(r, 2) == 0))
        def _():
            pltpu.async_copy(
                lat_hbm.at[gidx_v.at[r + 1, pl.ds(0, NSEL)]],
                grp1_v, sem1)

        @pl.when(jnp.logical_and(r + 1 < RPW, lax.rem(r, 2) == 1))
        def _():
            pltpu.async_copy(
                lat_hbm.at[gidx_v.at[r + 1, pl.ds(0, NSEL)]],
                grp0_v, sem0)

        lbv = jnp.full((16,), lb_v[pl.ds(r, 16)][0], jnp.float32)

        def scan_one(grp_v, sem):
            pltpu.make_async_copy(
                lat_hbm.at[gidx_v.at[r, pl.ds(0, NSEL)]], grp_v,
                sem).wait()

            def scan_g(g, cpos):
                for v in range(8):
                    xv = grp_v[g, pl.ds(v * 16, 16)]
                    msk = xv >= lbv
                    cs = plsc.cumsum(msk.astype(jnp.int32))
                    tgt = jnp.where(msk, cpos + cs - 1, CMAX - 16)
                    plsc.store_scatter(cw_v, [tgt], xv, mask=msk)
                    cpos = cpos + cs[15]
                return cpos

            return lax.fori_loop(0, NSEL, scan_g, jnp.int32(0))

        cpos = lax.cond(lax.rem(r, 2) == 0,
                        lambda: scan_one(grp0_v, sem0),
                        lambda: scan_one(grp1_v, sem1))
        cw_v[pl.ds(cpos, 16)] = neg16
        cw_v[pl.ds(cpos + 16, 16)] = neg16
        nv = (cpos + 15) // 16

        def ext3_body(j, _):
            a = cw_v[pl.ds(0, 16)]
            b = cw_v[pl.ds(16, 16)]
            c = cw_v[pl.ds(32, 16)]
            m = jnp.max(jnp.maximum(jnp.maximum(a, b), c))
            msp = jnp.full((16,), m, jnp.float32)
            cw_v[pl.ds(0, 16)] = jnp.where(a >= msp, -jnp.inf, a)
            cw_v[pl.ds(16, 16)] = jnp.where(b >= msp, -jnp.inf, b)
            cw_v[pl.ds(32, 16)] = jnp.where(c >= msp, -jnp.inf, c)
            return m

        def ext_body(j, _):
            def mx(v, acc):
                return jnp.maximum(acc, cw_v[pl.ds(v * 16, 16)])
            run = lax.fori_loop(0, nv, mx, neg16)
            m = jnp.max(run)
            msp = jnp.full((16,), m, jnp.float32)

            def rm(v, c):
                xv = cw_v[pl.ds(v * 16, 16)]
                cw_v[pl.ds(v * 16, 16)] = jnp.where(xv >= msp, -jnp.inf,
                                                    xv)
                return c
            lax.fori_loop(0, nv, rm, jnp.int32(0))
            return m

        tau = lax.cond(
            cpos <= 48,
            lambda: lax.fori_loop(0, TOPK, ext3_body, jnp.float32(0.0)),
            lambda: lax.fori_loop(0, TOPK, ext_body, jnp.float32(0.0)))
        tau_v[r] = jnp.full((16,), tau, jnp.float32)
        return 0

    lax.fori_loop(0, RPW, row_body, jnp.int32(0))
    pltpu.sync_copy(tau_v, tau_hbm.at[pl.ds(base, RPW)])


def _dec_body(lat_ref, tau_ref, wd_ref, bd_ref, out_ref):
    j = pl.program_id(1)

    @pl.when(j == 0)
    def _():
        out_ref[...] = jnp.broadcast_to(bd_ref[...], out_ref.shape)

    lat = lat_ref[...]
    tau = tau_ref[:, 0:1]
    masked = jnp.where(lat >= tau, lat, 0.0)
    out_ref[...] += jnp.dot(masked.astype(jnp.bfloat16),
                            wd_ref[...].astype(jnp.bfloat16),
                            preferred_element_type=jnp.float32)


@jax.jit
def kernel(x, W_enc, b_enc, W_dec, b_dec):
    be2 = b_enc.reshape(1, D_LATENT)
    bd2 = b_dec.reshape(1, D_MODEL)

    latents, M = pl.pallas_call(
        _enc_body,
        grid=(N_TOKENS // BR_ENC, D_LATENT // BC_ENC),
        in_specs=[
            pl.BlockSpec((BR_ENC, D_MODEL), lambda i, j: (i, 0)),
            pl.BlockSpec((D_MODEL, BC_ENC), lambda i, j: (0, j)),
            pl.BlockSpec((1, BC_ENC), lambda i, j: (0, j)),
        ],
        out_specs=[
            pl.BlockSpec((BR_ENC, BC_ENC), lambda i, j: (i, j)),
            pl.BlockSpec((1, BR_ENC, GPB), lambda i, j: (j, i, 0)),
        ],
        out_shape=[
            jax.ShapeDtypeStruct((N_TOKENS, D_LATENT), jnp.float32),
            jax.ShapeDtypeStruct((D_LATENT // BC_ENC, N_TOKENS, GPB),
                                 jnp.float32),
        ],
        compiler_params=pltpu.CompilerParams(
            dimension_semantics=("parallel", "parallel")),
    )(x, W_enc, be2)

    M = jnp.transpose(M, (1, 0, 2)).reshape(N_TOKENS, NGRP)

    gidx, lb = pl.pallas_call(
        _sel_body,
        grid=(N_TOKENS // BR_SEL,),
        in_specs=[pl.BlockSpec((BR_SEL, NGRP), lambda i: (i, 0))],
        out_specs=[
            pl.BlockSpec((BR_SEL, NSEL_PAD), lambda i: (i, 0)),
            pl.BlockSpec((BR_SEL, 1), lambda i: (i, 0)),
        ],
        out_shape=[
            jax.ShapeDtypeStruct((N_TOKENS, NSEL_PAD), jnp.int32),
            jax.ShapeDtypeStruct((N_TOKENS, 1), jnp.float32),
        ],
        compiler_params=pltpu.CompilerParams(
            dimension_semantics=("parallel",)),
    )(M)

    lat_flat = latents.reshape(N_TOKENS * NGRP, 128)
    lb_flat = lb.reshape(N_TOKENS)

    tau = pl.kernel(
        _sc_body,
        mesh=plsc.VectorSubcoreMesh(core_axis_name="c",
                                    subcore_axis_name="s"),
        compiler_params=pltpu.CompilerParams(needs_layout_passes=False),
        out_type=jax.ShapeDtypeStruct((N_TOKENS, 16), jnp.float32),
        scratch_types=[
            pltpu.VMEM((RPW, NSEL_PAD), jnp.int32),
            pltpu.VMEM((RPW + 16,), jnp.float32),
            pltpu.VMEM((NSEL, 128), jnp.float32),
            pltpu.VMEM((NSEL, 128), jnp.float32),
            pltpu.VMEM((CMAX,), jnp.float32),
            pltpu.VMEM((RPW, 16), jnp.float32),
            pltpu.SemaphoreType.DMA,
            pltpu.SemaphoreType.DMA,
        ],
    )(lat_flat, gidx, lb_flat)

    tau2 = tau[:, 0:1]

    recons = pl.pallas_call(
        _dec_body,
        grid=(N_TOKENS // BR_DEC, D_LATENT // BK_DEC),
        in_specs=[
            pl.BlockSpec((BR_DEC, BK_DEC), lambda i, j: (i, j)),
            pl.BlockSpec((BR_DEC, 1), lambda i, j: (i, 0)),
            pl.BlockSpec((BK_DEC, D_MODEL), lambda i, j: (j, 0)),
            pl.BlockSpec((1, D_MODEL), lambda i, j: (0, 0)),
        ],
        out_specs=pl.BlockSpec((BR_DEC, D_MODEL), lambda i, j: (i, 0)),
        out_shape=jax.ShapeDtypeStruct((N_TOKENS, D_MODEL), jnp.float32),
        compiler_params=pltpu.CompilerParams(
            dimension_semantics=("parallel", "arbitrary")),
    )(latents, tau2, W_dec, bd2)

    return recons
